# baseline (speedup 1.0000x reference)
import functools

import jax
import jax.numpy as jnp
from jax import lax
from jax.experimental import pallas as pl
from jax.experimental.pallas import tpu as pltpu
from jax.experimental.pallas import tpu_sc as plsc

B, N, F = 16, 10000, 128
NC, NS = 2, 16
NW = NC * NS
WPB = NW // B
ROWS = N // WPB
CHUNK = 200
NCHUNK = ROWS // CHUNK
NV = F // 16

_mesh = plsc.VectorSubcoreMesh(core_axis_name="c", subcore_axis_name="s")


@functools.partial(
    pl.kernel,
    out_type=jax.ShapeDtypeStruct((WPB * B * F,), jnp.float32),
    mesh=_mesh,
    scratch_types=[
        pltpu.VMEM((2, CHUNK, F), jnp.float32),
        pltpu.VMEM((F,), jnp.float32),
        pltpu.SemaphoreType.DMA,
        pltpu.SemaphoreType.DMA,
    ],
)
def _pool(x_hbm, out_hbm, buf, accv, sem0, sem1):
    c = lax.axis_index("c")
    s = lax.axis_index("s")
    wid = c * NS + s
    b = wid // WPB
    h = wid % WPB
    row0 = h * ROWS
    sems = (sem0, sem1)

    def start(i, slot):
        pltpu.make_async_copy(
            x_hbm.at[b, pl.ds(row0 + i * CHUNK, CHUNK), :],
            buf.at[slot], sems[slot]).start()

    def wait(slot):
        pltpu.make_async_copy(
            x_hbm.at[b, pl.ds(0, CHUNK), :],
            buf.at[slot], sems[slot]).wait()

    def consume(slot, acc):
        @plsc.parallel_loop(0, CHUNK, carry=acc, unroll=4)
        def row_body(r, a):
            a = list(a)
            for j in range(NV):
                a[j] = jnp.maximum(a[j], buf[slot, r, pl.ds(j * 16, 16)])
            return tuple(a)

        return row_body

    start(0, 0)
    start(1, 1)

    acc0 = tuple(jnp.full((16,), -jnp.inf, jnp.float32) for _ in range(NV))

    def pair_body(g, acc):
        for slot in range(2):
            i = 2 * g + slot
            wait(slot)
            acc = consume(slot, acc)
            nxt = i + 2

            @pl.when(nxt < NCHUNK)
            def _():
                start(nxt, slot)
        return acc

    acc = lax.fori_loop(0, NCHUNK // 2, pair_body, acc0)
    if NCHUNK % 2:
        wait(0)
        acc = consume(0, acc)

    for j in range(NV):
        accv[pl.ds(j * 16, 16)] = acc[j]
    pltpu.sync_copy(accv, out_hbm.at[pl.ds((h * B + b) * F, F)])


TC_CHUNK = 1000


def _tc_body(x_ref, o_ref):
    n = pl.program_id(1)

    @pl.when(n == 0)
    def _():
        o_ref[...] = jnp.full_like(o_ref, -jnp.inf)

    o_ref[...] = jnp.maximum(o_ref[...], jnp.max(x_ref[...], axis=1,
                                                 keepdims=True))


def _tc_pool(x, row_lo, row_hi):
    nblk = (row_hi - row_lo) // TC_CHUNK
    out = pl.pallas_call(
        _tc_body,
        grid=(B, nblk),
        in_specs=[pl.BlockSpec(
            (1, TC_CHUNK, F),
            lambda b, n: (b, row_lo // TC_CHUNK + n, 0))],
        out_specs=pl.BlockSpec((1, 1, F), lambda b, n: (b, 0, 0)),
        out_shape=jax.ShapeDtypeStruct((B, 1, F), jnp.float32),
    )(x)
    return out.reshape(B, F)


def kernel(x):
    return _tc_pool(x, 0, N)


# speedup vs baseline: 1.5928x; 1.5928x over previous
import functools

import jax
import jax.numpy as jnp
from jax import lax
from jax.experimental import pallas as pl
from jax.experimental.pallas import tpu as pltpu
from jax.experimental.pallas import tpu_sc as plsc

B, N, F = 16, 10000, 128
NC, NS = 2, 16
NW = NC * NS
WPB = NW // B
ROWS = N // WPB
CHUNK = 200
NCHUNK = ROWS // CHUNK
NV = F // 16

_mesh = plsc.VectorSubcoreMesh(core_axis_name="c", subcore_axis_name="s")


@functools.partial(
    pl.kernel,
    out_type=jax.ShapeDtypeStruct((WPB * B * F,), jnp.float32),
    mesh=_mesh,
    scratch_types=[
        pltpu.VMEM((2, CHUNK, F), jnp.float32),
        pltpu.VMEM((F,), jnp.float32),
        pltpu.SemaphoreType.DMA,
        pltpu.SemaphoreType.DMA,
    ],
)
def _pool(x_hbm, out_hbm, buf, accv, sem0, sem1):
    c = lax.axis_index("c")
    s = lax.axis_index("s")
    wid = c * NS + s
    b = wid // WPB
    h = wid % WPB
    row0 = h * ROWS
    sems = (sem0, sem1)

    def start(i, slot):
        pltpu.make_async_copy(
            x_hbm.at[b, pl.ds(row0 + i * CHUNK, CHUNK), :],
            buf.at[slot], sems[slot]).start()

    def wait(slot):
        pltpu.make_async_copy(
            x_hbm.at[b, pl.ds(0, CHUNK), :],
            buf.at[slot], sems[slot]).wait()

    def consume(slot, acc):
        @plsc.parallel_loop(0, CHUNK, carry=acc, unroll=4)
        def row_body(r, a):
            a = list(a)
            for j in range(NV):
                a[j] = jnp.maximum(a[j], buf[slot, r, pl.ds(j * 16, 16)])
            return tuple(a)

        return row_body

    start(0, 0)
    start(1, 1)

    acc0 = tuple(jnp.full((16,), -jnp.inf, jnp.float32) for _ in range(NV))

    def pair_body(g, acc):
        for slot in range(2):
            i = 2 * g + slot
            wait(slot)
            acc = consume(slot, acc)
            nxt = i + 2

            @pl.when(nxt < NCHUNK)
            def _():
                start(nxt, slot)
        return acc

    acc = lax.fori_loop(0, NCHUNK // 2, pair_body, acc0)
    if NCHUNK % 2:
        wait(0)
        acc = consume(0, acc)

    for j in range(NV):
        accv[pl.ds(j * 16, 16)] = acc[j]
    pltpu.sync_copy(accv, out_hbm.at[pl.ds((h * B + b) * F, F)])


TC_CHUNK = 2000


def _tc_body(x_ref, o_ref):
    n = pl.program_id(1)

    @pl.when(n == 0)
    def _():
        o_ref[...] = jnp.full_like(o_ref, -jnp.inf)

    o_ref[...] = jnp.maximum(o_ref[...], jnp.max(x_ref[...], axis=1,
                                                 keepdims=True))


def _tc_pool(x, row_lo, row_hi):
    nblk = (row_hi - row_lo) // TC_CHUNK
    out = pl.pallas_call(
        _tc_body,
        grid=(B, nblk),
        in_specs=[pl.BlockSpec(
            (1, TC_CHUNK, F),
            lambda b, n: (b, row_lo // TC_CHUNK + n, 0))],
        out_specs=pl.BlockSpec((1, 1, F), lambda b, n: (b, 0, 0)),
        out_shape=jax.ShapeDtypeStruct((B, 1, F), jnp.float32),
    )(x)
    return out.reshape(B, F)


def kernel(x):
    return _tc_pool(x, 0, N)


# speedup vs baseline: 2.4484x; 1.5372x over previous
import functools

import jax
import jax.numpy as jnp
from jax import lax
from jax.experimental import pallas as pl
from jax.experimental.pallas import tpu as pltpu
from jax.experimental.pallas import tpu_sc as plsc

B, N, F = 16, 10000, 128
NC, NS = 2, 16
NW = NC * NS
WPB = NW // B
ROWS = N // WPB
CHUNK = 200
NCHUNK = ROWS // CHUNK
NV = F // 16

_mesh = plsc.VectorSubcoreMesh(core_axis_name="c", subcore_axis_name="s")


@functools.partial(
    pl.kernel,
    out_type=jax.ShapeDtypeStruct((WPB * B * F,), jnp.float32),
    mesh=_mesh,
    scratch_types=[
        pltpu.VMEM((2, CHUNK, F), jnp.float32),
        pltpu.VMEM((F,), jnp.float32),
        pltpu.SemaphoreType.DMA,
        pltpu.SemaphoreType.DMA,
    ],
)
def _pool(x_hbm, out_hbm, buf, accv, sem0, sem1):
    c = lax.axis_index("c")
    s = lax.axis_index("s")
    wid = c * NS + s
    b = wid // WPB
    h = wid % WPB
    row0 = h * ROWS
    sems = (sem0, sem1)

    def start(i, slot):
        pltpu.make_async_copy(
            x_hbm.at[b, pl.ds(row0 + i * CHUNK, CHUNK), :],
            buf.at[slot], sems[slot]).start()

    def wait(slot):
        pltpu.make_async_copy(
            x_hbm.at[b, pl.ds(0, CHUNK), :],
            buf.at[slot], sems[slot]).wait()

    def consume(slot, acc):
        @plsc.parallel_loop(0, CHUNK, carry=acc, unroll=4)
        def row_body(r, a):
            a = list(a)
            for j in range(NV):
                a[j] = jnp.maximum(a[j], buf[slot, r, pl.ds(j * 16, 16)])
            return tuple(a)

        return row_body

    start(0, 0)
    start(1, 1)

    acc0 = tuple(jnp.full((16,), -jnp.inf, jnp.float32) for _ in range(NV))

    def pair_body(g, acc):
        for slot in range(2):
            i = 2 * g + slot
            wait(slot)
            acc = consume(slot, acc)
            nxt = i + 2

            @pl.when(nxt < NCHUNK)
            def _():
                start(nxt, slot)
        return acc

    acc = lax.fori_loop(0, NCHUNK // 2, pair_body, acc0)
    if NCHUNK % 2:
        wait(0)
        acc = consume(0, acc)

    for j in range(NV):
        accv[pl.ds(j * 16, 16)] = acc[j]
    pltpu.sync_copy(accv, out_hbm.at[pl.ds((h * B + b) * F, F)])


TC_CHUNK = 5000


def _tc_body(x_ref, o_ref):
    n = pl.program_id(1)

    @pl.when(n == 0)
    def _():
        o_ref[...] = jnp.full_like(o_ref, -jnp.inf)

    o_ref[...] = jnp.maximum(o_ref[...], jnp.max(x_ref[...], axis=1,
                                                 keepdims=True))


def _tc_pool(x, row_lo, row_hi):
    nblk = (row_hi - row_lo) // TC_CHUNK
    out = pl.pallas_call(
        _tc_body,
        grid=(B, nblk),
        in_specs=[pl.BlockSpec(
            (1, TC_CHUNK, F),
            lambda b, n: (b, row_lo // TC_CHUNK + n, 0))],
        out_specs=pl.BlockSpec((1, 1, F), lambda b, n: (b, 0, 0)),
        out_shape=jax.ShapeDtypeStruct((B, 1, F), jnp.float32),
    )(x)
    return out.reshape(B, F)


def kernel(x):
    return _tc_pool(x, 0, N)


# speedup vs baseline: 3.0395x; 1.2414x over previous
import functools

import jax
import jax.numpy as jnp
from jax import lax
from jax.experimental import pallas as pl
from jax.experimental.pallas import tpu as pltpu
from jax.experimental.pallas import tpu_sc as plsc

B, N, F = 16, 10000, 128
NC, NS = 2, 16
NW = NC * NS
WPB = NW // B
ROWS = N // WPB
CHUNK = 200
NCHUNK = ROWS // CHUNK
NV = F // 16

_mesh = plsc.VectorSubcoreMesh(core_axis_name="c", subcore_axis_name="s")


@functools.partial(
    pl.kernel,
    out_type=jax.ShapeDtypeStruct((WPB * B * F,), jnp.float32),
    mesh=_mesh,
    scratch_types=[
        pltpu.VMEM((2, CHUNK, F), jnp.float32),
        pltpu.VMEM((F,), jnp.float32),
        pltpu.SemaphoreType.DMA,
        pltpu.SemaphoreType.DMA,
    ],
)
def _pool(x_hbm, out_hbm, buf, accv, sem0, sem1):
    c = lax.axis_index("c")
    s = lax.axis_index("s")
    wid = c * NS + s
    b = wid // WPB
    h = wid % WPB
    row0 = h * ROWS
    sems = (sem0, sem1)

    def start(i, slot):
        pltpu.make_async_copy(
            x_hbm.at[b, pl.ds(row0 + i * CHUNK, CHUNK), :],
            buf.at[slot], sems[slot]).start()

    def wait(slot):
        pltpu.make_async_copy(
            x_hbm.at[b, pl.ds(0, CHUNK), :],
            buf.at[slot], sems[slot]).wait()

    def consume(slot, acc):
        @plsc.parallel_loop(0, CHUNK, carry=acc, unroll=4)
        def row_body(r, a):
            a = list(a)
            for j in range(NV):
                a[j] = jnp.maximum(a[j], buf[slot, r, pl.ds(j * 16, 16)])
            return tuple(a)

        return row_body

    start(0, 0)
    start(1, 1)

    acc0 = tuple(jnp.full((16,), -jnp.inf, jnp.float32) for _ in range(NV))

    def pair_body(g, acc):
        for slot in range(2):
            i = 2 * g + slot
            wait(slot)
            acc = consume(slot, acc)
            nxt = i + 2

            @pl.when(nxt < NCHUNK)
            def _():
                start(nxt, slot)
        return acc

    acc = lax.fori_loop(0, NCHUNK // 2, pair_body, acc0)
    if NCHUNK % 2:
        wait(0)
        acc = consume(0, acc)

    for j in range(NV):
        accv[pl.ds(j * 16, 16)] = acc[j]
    pltpu.sync_copy(accv, out_hbm.at[pl.ds((h * B + b) * F, F)])


TC_CHUNK = 10000


def _tc_body(x_ref, o_ref):
    n = pl.program_id(1)

    @pl.when(n == 0)
    def _():
        o_ref[...] = jnp.full_like(o_ref, -jnp.inf)

    o_ref[...] = jnp.maximum(o_ref[...], jnp.max(x_ref[...], axis=1,
                                                 keepdims=True))


def _tc_pool(x, row_lo, row_hi):
    nblk = (row_hi - row_lo) // TC_CHUNK
    out = pl.pallas_call(
        _tc_body,
        grid=(B, nblk),
        in_specs=[pl.BlockSpec(
            (1, TC_CHUNK, F),
            lambda b, n: (b, row_lo // TC_CHUNK + n, 0))],
        out_specs=pl.BlockSpec((1, 1, F), lambda b, n: (b, 0, 0)),
        out_shape=jax.ShapeDtypeStruct((B, 1, F), jnp.float32),
    )(x)
    return out.reshape(B, F)


def kernel(x):
    return _tc_pool(x, 0, N)


# speedup vs baseline: 3.7705x; 1.2405x over previous
import functools

import jax
import jax.numpy as jnp
from jax import lax
from jax.experimental import pallas as pl
from jax.experimental.pallas import tpu as pltpu
from jax.experimental.pallas import tpu_sc as plsc

B, N, F = 16, 10000, 128
NC, NS = 2, 16
NW = NC * NS
WPB = NW // B
ROWS = N // WPB
CHUNK = 200
NCHUNK = ROWS // CHUNK
NV = F // 16

_mesh = plsc.VectorSubcoreMesh(core_axis_name="c", subcore_axis_name="s")


@functools.partial(
    pl.kernel,
    out_type=jax.ShapeDtypeStruct((WPB * B * F,), jnp.float32),
    mesh=_mesh,
    scratch_types=[
        pltpu.VMEM((2, CHUNK, F), jnp.float32),
        pltpu.VMEM((F,), jnp.float32),
        pltpu.SemaphoreType.DMA,
        pltpu.SemaphoreType.DMA,
    ],
)
def _pool(x_hbm, out_hbm, buf, accv, sem0, sem1):
    c = lax.axis_index("c")
    s = lax.axis_index("s")
    wid = c * NS + s
    b = wid // WPB
    h = wid % WPB
    row0 = h * ROWS
    sems = (sem0, sem1)

    def start(i, slot):
        pltpu.make_async_copy(
            x_hbm.at[b, pl.ds(row0 + i * CHUNK, CHUNK), :],
            buf.at[slot], sems[slot]).start()

    def wait(slot):
        pltpu.make_async_copy(
            x_hbm.at[b, pl.ds(0, CHUNK), :],
            buf.at[slot], sems[slot]).wait()

    def consume(slot, acc):
        @plsc.parallel_loop(0, CHUNK, carry=acc, unroll=4)
        def row_body(r, a):
            a = list(a)
            for j in range(NV):
                a[j] = jnp.maximum(a[j], buf[slot, r, pl.ds(j * 16, 16)])
            return tuple(a)

        return row_body

    start(0, 0)
    start(1, 1)

    acc0 = tuple(jnp.full((16,), -jnp.inf, jnp.float32) for _ in range(NV))

    def pair_body(g, acc):
        for slot in range(2):
            i = 2 * g + slot
            wait(slot)
            acc = consume(slot, acc)
            nxt = i + 2

            @pl.when(nxt < NCHUNK)
            def _():
                start(nxt, slot)
        return acc

    acc = lax.fori_loop(0, NCHUNK // 2, pair_body, acc0)
    if NCHUNK % 2:
        wait(0)
        acc = consume(0, acc)

    for j in range(NV):
        accv[pl.ds(j * 16, 16)] = acc[j]
    pltpu.sync_copy(accv, out_hbm.at[pl.ds((h * B + b) * F, F)])


TC_CHUNK = 10000
TC_BB = 2


def _tc_body(x_ref, o_ref):
    n = pl.program_id(1)

    @pl.when(n == 0)
    def _():
        o_ref[...] = jnp.full_like(o_ref, -jnp.inf)

    o_ref[...] = jnp.maximum(o_ref[...], jnp.max(x_ref[...], axis=1,
                                                 keepdims=True))


def _tc_pool(x, row_lo, row_hi):
    nblk = (row_hi - row_lo) // TC_CHUNK
    out = pl.pallas_call(
        _tc_body,
        grid=(B // TC_BB, nblk),
        in_specs=[pl.BlockSpec(
            (TC_BB, TC_CHUNK, F),
            lambda b, n: (b, row_lo // TC_CHUNK + n, 0))],
        out_specs=pl.BlockSpec((TC_BB, 1, F), lambda b, n: (b, 0, 0)),
        out_shape=jax.ShapeDtypeStruct((B, 1, F), jnp.float32),
    )(x)
    return out.reshape(B, F)


def kernel(x):
    return _tc_pool(x, 0, N)


# speedup vs baseline: 3.7756x; 1.0013x over previous
import functools

import jax
import jax.numpy as jnp
from jax import lax
from jax.experimental import pallas as pl
from jax.experimental.pallas import tpu as pltpu
from jax.experimental.pallas import tpu_sc as plsc

B, N, F = 16, 10000, 128
NC, NS = 2, 16
NW = NC * NS
WPB = NW // B
ROWS = N // WPB
CHUNK = 200
NCHUNK = ROWS // CHUNK
NV = F // 16

_mesh = plsc.VectorSubcoreMesh(core_axis_name="c", subcore_axis_name="s")


@functools.partial(
    pl.kernel,
    out_type=jax.ShapeDtypeStruct((WPB * B * F,), jnp.float32),
    mesh=_mesh,
    scratch_types=[
        pltpu.VMEM((2, CHUNK, F), jnp.float32),
        pltpu.VMEM((F,), jnp.float32),
        pltpu.SemaphoreType.DMA,
        pltpu.SemaphoreType.DMA,
    ],
)
def _pool(x_hbm, out_hbm, buf, accv, sem0, sem1):
    c = lax.axis_index("c")
    s = lax.axis_index("s")
    wid = c * NS + s
    b = wid // WPB
    h = wid % WPB
    row0 = h * ROWS
    sems = (sem0, sem1)

    def start(i, slot):
        pltpu.make_async_copy(
            x_hbm.at[b, pl.ds(row0 + i * CHUNK, CHUNK), :],
            buf.at[slot], sems[slot]).start()

    def wait(slot):
        pltpu.make_async_copy(
            x_hbm.at[b, pl.ds(0, CHUNK), :],
            buf.at[slot], sems[slot]).wait()

    def consume(slot, acc):
        @plsc.parallel_loop(0, CHUNK, carry=acc, unroll=4)
        def row_body(r, a):
            a = list(a)
            for j in range(NV):
                a[j] = jnp.maximum(a[j], buf[slot, r, pl.ds(j * 16, 16)])
            return tuple(a)

        return row_body

    start(0, 0)
    start(1, 1)

    acc0 = tuple(jnp.full((16,), -jnp.inf, jnp.float32) for _ in range(NV))

    def pair_body(g, acc):
        for slot in range(2):
            i = 2 * g + slot
            wait(slot)
            acc = consume(slot, acc)
            nxt = i + 2

            @pl.when(nxt < NCHUNK)
            def _():
                start(nxt, slot)
        return acc

    acc = lax.fori_loop(0, NCHUNK // 2, pair_body, acc0)
    if NCHUNK % 2:
        wait(0)
        acc = consume(0, acc)

    for j in range(NV):
        accv[pl.ds(j * 16, 16)] = acc[j]
    pltpu.sync_copy(accv, out_hbm.at[pl.ds((h * B + b) * F, F)])


TC_CHUNK = 10000
TC_BB = 4


def _tc_body(x_ref, o_ref):
    n = pl.program_id(1)

    @pl.when(n == 0)
    def _():
        o_ref[...] = jnp.full_like(o_ref, -jnp.inf)

    o_ref[...] = jnp.maximum(o_ref[...], jnp.max(x_ref[...], axis=1,
                                                 keepdims=True))


def _tc_pool(x, row_lo, row_hi):
    nblk = (row_hi - row_lo) // TC_CHUNK
    out = pl.pallas_call(
        _tc_body,
        grid=(B // TC_BB, nblk),
        in_specs=[pl.BlockSpec(
            (TC_BB, TC_CHUNK, F),
            lambda b, n: (b, row_lo // TC_CHUNK + n, 0))],
        out_specs=pl.BlockSpec((TC_BB, 1, F), lambda b, n: (b, 0, 0)),
        out_shape=jax.ShapeDtypeStruct((B, 1, F), jnp.float32),
    )(x)
    return out.reshape(B, F)


def kernel(x):
    return _tc_pool(x, 0, N)
